# baseline (speedup 1.0000x reference)
import functools

import jax
import jax.numpy as jnp
from jax import lax
from jax.experimental import pallas as pl
from jax.experimental.pallas import tpu as pltpu

HEAD_PAD = 128
BT = 32


def _make_body(H, W, Bt):
    HW = H * W

    def body(x_ref, wconvT_ref, bconv_ref, wsedT_ref, bsedT_ref,
             wascT_ref, bascT_ref, dseT_ref, ye_ref, ys_ref, ee_ref):
        pos = lax.broadcasted_iota(jnp.int32, (1, HW), 1)
        h_idx = pos // W
        w_idx = pos % W
        taps = []
        for dy in (-1, 0, 1):
            for dx in (-1, 0, 1):
                conds = []
                if dy < 0:
                    conds.append(h_idx >= -dy)
                if dy > 0:
                    conds.append(h_idx < H - dy)
                if dx < 0:
                    conds.append(w_idx >= -dx)
                if dx > 0:
                    conds.append(w_idx < W - dx)
                mask = None
                if conds:
                    mask = conds[0]
                    for c in conds[1:]:
                        mask = jnp.logical_and(mask, c)
                shift = (-(dy * W + dx)) % HW
                taps.append((shift, mask))

        wconvT = wconvT_ref[...]
        bconv = bconv_ref[...]
        inv_hw = 1.0 / float(HW)
        zero = jnp.zeros((), jnp.bfloat16)

        pooled_cols = []
        for b in range(Bt):
            xb = x_ref[b].astype(jnp.bfloat16)
            cols = []
            for shift, mask in taps:
                t = xb if shift == 0 else pltpu.roll(xb, shift=shift, axis=1)
                if mask is not None:
                    t = jnp.where(mask, t, zero)
                cols.append(t)
            patches = jnp.concatenate(cols, axis=0)
            rT = jnp.dot(wconvT, patches,
                         preferred_element_type=jnp.float32)
            rT = jnp.maximum(rT + bconv, 0.0)
            pooled_cols.append(jnp.sum(rT, axis=1, keepdims=True))

        pooledT = jnp.concatenate(pooled_cols, axis=1) * inv_hw

        def sigmoid(z):
            return 0.5 * (jnp.tanh(0.5 * z) + 1.0)

        eeT = sigmoid(jnp.dot(wsedT_ref[...], pooledT,
                              preferred_element_type=jnp.float32)
                      + bsedT_ref[...])
        logits = jnp.dot(wascT_ref[...], pooledT,
                         preferred_element_type=jnp.float32) + bascT_ref[...]
        ex = jnp.exp(logits - jnp.max(logits, axis=0, keepdims=True))
        ysT = ex / jnp.sum(ex, axis=0, keepdims=True)
        mseT = sigmoid(jnp.dot(dseT_ref[...], ysT,
                               preferred_element_type=jnp.float32))
        yeT = eeT * mseT

        ye_ref[...] = yeT.T
        ys_ref[...] = ysT.T
        ee_ref[...] = eeT.T

    return body


def _full_spec(arr):
    n = arr.ndim
    return pl.BlockSpec(arr.shape, lambda *_: (0,) * n)


@functools.partial(jax.jit, static_argnames=("sed_class", "asc_class"))
def _forward(x, wconvT, bconv, wsed, bsed, wasc, basc, dse, *,
             sed_class, asc_class):
    B, Cin, H, W = x.shape
    HW = H * W
    x_flat = x.reshape(B, Cin, HW).astype(jnp.float32)

    steps = pl.cdiv(B, BT)
    B_pad = steps * BT
    if B_pad != B:
        x_flat = jnp.concatenate(
            [x_flat, jnp.zeros((B_pad - B, Cin, HW), x_flat.dtype)], axis=0)

    wconvT_bf = wconvT.astype(jnp.bfloat16)
    wsedT = wsed.T
    bsedT = bsed.T
    wascT = wasc.T
    bascT = basc.T
    dseT = dse.T

    out_shape = jax.ShapeDtypeStruct((B_pad, HEAD_PAD), jnp.float32)
    out_spec = pl.BlockSpec((BT, HEAD_PAD), lambda i: (i, 0))

    y_e, y_s, e_e = pl.pallas_call(
        _make_body(H, W, BT),
        out_shape=(out_shape, out_shape, out_shape),
        grid=(steps,),
        in_specs=[
            pl.BlockSpec((BT, Cin, HW), lambda i: (i, 0, 0)),
            _full_spec(wconvT_bf), _full_spec(bconv),
            _full_spec(wsedT), _full_spec(bsedT),
            _full_spec(wascT), _full_spec(bascT),
            _full_spec(dseT),
        ],
        out_specs=(out_spec, out_spec, out_spec),
        compiler_params=pltpu.CompilerParams(
            dimension_semantics=("parallel",)),
    )(x_flat, wconvT_bf, bconv, wsedT, bsedT, wascT, bascT, dseT)

    return (y_e[:B, :sed_class], y_s[:B, :asc_class], e_e[:B, :sed_class])


def kernel(x, wconvT, bconv, wsed, bsed, wasc, basc, dse):
    return _forward(x, wconvT, bconv, wsed, bsed, wasc, basc, dse,
                    sed_class=64, asc_class=32)


# speedup vs baseline: 1.7166x; 1.7166x over previous
import functools

import jax
import jax.numpy as jnp
from jax import lax
from jax.experimental import pallas as pl
from jax.experimental.pallas import tpu as pltpu

HEAD_PAD = 128
BT = 16
LP = 128


def _make_body(H, W, Bt, Cin):
    HW = H * W
    HWP = HW + 2 * LP

    def body(x_ref, p_ref, out_ref):
        x2 = x_ref[...].reshape(Bt * Cin, HW).astype(jnp.bfloat16)
        zpad = jnp.zeros((Bt * Cin, LP), jnp.bfloat16)
        xp = jnp.concatenate([zpad, x2, zpad], axis=1)

        pos = lax.broadcasted_iota(jnp.int32, (1, HWP), 1)
        w_idx = (pos - LP) % W
        zero = jnp.zeros((), jnp.bfloat16)
        zm1 = jnp.where(w_idx >= 1, pltpu.roll(xp, shift=1, axis=1), zero)
        zp1 = jnp.where(w_idx <= W - 2,
                        pltpu.roll(xp, shift=HWP - 1, axis=1), zero)
        mids = (zm1, xp, zp1)
        ups = tuple(pltpu.roll(z, shift=W, axis=1) for z in mids)
        downs = tuple(pltpu.roll(z, shift=HWP - W, axis=1) for z in mids)
        groups = ups + mids + downs

        wfull = p_ref[:, 0:73].astype(jnp.bfloat16)
        ones_row = jnp.ones((1, HW), jnp.bfloat16)
        inv_hw = 1.0 / float(HW)

        pooled_cols = []
        for b in range(Bt):
            rows = [g[b * Cin:(b + 1) * Cin, LP:LP + HW] for g in groups]
            patches = jnp.concatenate(rows + [ones_row], axis=0)
            rT = jnp.dot(wfull, patches,
                         preferred_element_type=jnp.float32)
            rT = jnp.maximum(rT, 0.0)
            pooled_cols.append(jnp.sum(rT, axis=1, keepdims=True))

        pooledT = jnp.concatenate(pooled_cols, axis=1) * inv_hw

        def sigmoid(z):
            return 0.5 * (jnp.tanh(0.5 * z) + 1.0)

        eeT = sigmoid(jnp.dot(p_ref[:, 128:256], pooledT,
                              preferred_element_type=jnp.float32)
                      + p_ref[:, 512:513])
        logits = (jnp.dot(p_ref[:, 256:384], pooledT,
                          preferred_element_type=jnp.float32)
                  + p_ref[:, 513:514])
        ex = jnp.exp(logits - jnp.max(logits, axis=0, keepdims=True))
        ysT = ex / jnp.sum(ex, axis=0, keepdims=True)
        mseT = sigmoid(jnp.dot(p_ref[:, 384:512], ysT,
                               preferred_element_type=jnp.float32))
        yeT = eeT * mseT

        out_ref[...] = jnp.concatenate([yeT.T, ysT.T, eeT.T], axis=1)

    return body


@functools.partial(jax.jit, static_argnames=("sed_class", "asc_class"))
def _forward(x, wconvT, bconv, wsed, bsed, wasc, basc, dse, *,
             sed_class, asc_class):
    B, Cin, H, W = x.shape
    HW = H * W
    x_flat = x.reshape(B, Cin, HW).astype(jnp.float32)

    steps = pl.cdiv(B, BT)
    B_pad = steps * BT
    if B_pad != B:
        x_flat = jnp.concatenate(
            [x_flat, jnp.zeros((B_pad - B, Cin, HW), x_flat.dtype)], axis=0)

    cout = wconvT.shape[0]
    packed = jnp.concatenate([
        wconvT,
        bconv,
        jnp.zeros((cout, 55), jnp.float32),
        wsed.T, wasc.T, dse.T,
        bsed.T, basc.T,
        jnp.zeros((cout, 126), jnp.float32),
    ], axis=1)

    out_shape = jax.ShapeDtypeStruct((B_pad, 3 * HEAD_PAD), jnp.float32)
    out_spec = pl.BlockSpec((BT, 3 * HEAD_PAD), lambda i: (i, 0))

    out = pl.pallas_call(
        _make_body(H, W, BT, Cin),
        out_shape=out_shape,
        grid=(steps,),
        in_specs=[
            pl.BlockSpec((BT, Cin, HW), lambda i: (i, 0, 0)),
            pl.BlockSpec(packed.shape, lambda i: (0, 0)),
        ],
        out_specs=out_spec,
        compiler_params=pltpu.CompilerParams(
            dimension_semantics=("parallel",)),
    )(x_flat, packed)

    return (out[:B, 0:sed_class],
            out[:B, HEAD_PAD:HEAD_PAD + asc_class],
            out[:B, 2 * HEAD_PAD:2 * HEAD_PAD + sed_class])


def kernel(x, wconvT, bconv, wsed, bsed, wasc, basc, dse):
    return _forward(x, wconvT, bconv, wsed, bsed, wasc, basc, dse,
                    sed_class=64, asc_class=32)
